# Initial kernel scaffold; baseline (speedup 1.0000x reference)
import jax
import jax.numpy as jnp
from jax.experimental import pallas as pl


def kernel(x, edge_index, W_fc, b_fc, W1, W2):
    raise NotImplementedError("write your pallas kernel here")


# speedup vs baseline: 21.7906x; 21.7906x over previous
import functools
import math

import jax
import jax.numpy as jnp
from jax import lax
from jax.experimental import pallas as pl
from jax.experimental.pallas import tpu as pltpu
from jax.experimental.pallas import tpu_sc as plsc

N = 10000
E = 320000
D = 128
LAMDA = 0.5
ALPHA = 0.1
TH1 = math.log(LAMDA / 1.0 + 1.0)
TH2 = math.log(LAMDA / 2.0 + 1.0)

NC = 2
NS = 16
NCH = 80
C = 125
NP = 10240
RP = NP // NS

_sc_mesh = plsc.VectorSubcoreMesh(
    core_axis_name="c", subcore_axis_name="s", num_cores=NC, num_subcores=NS
)


@functools.partial(
    pl.kernel,
    out_type=jax.ShapeDtypeStruct((NC, NP, 16), jnp.float32),
    mesh=_sc_mesh,
    scratch_types=[
        pltpu.VMEM((NCH, C), jnp.int32),
        pltpu.VMEM((C, 16), jnp.float32),
        pltpu.VMEM_SHARED((NP, 16), jnp.float32),
    ],
)
def _deg_kernel(dst_hbm, ones_hbm, zeros16_hbm, out_hbm, dstv, onesv, accd):
    c = lax.axis_index("c")
    s = lax.axis_index("s")
    pltpu.sync_copy(dst_hbm.at[c, s], dstv)
    pltpu.sync_copy(ones_hbm, onesv)
    pltpu.sync_copy(zeros16_hbm.at[pl.ds(s * RP, RP)], accd.at[pl.ds(s * RP, RP)])
    plsc.subcore_barrier()

    def body(g, carry):
        pltpu.sync_copy(onesv, accd.at[dstv.at[g]], add=True)
        return carry

    lax.fori_loop(0, NCH, body, 0)
    plsc.subcore_barrier()
    pltpu.sync_copy(accd.at[pl.ds(s * RP, RP)], out_hbm.at[c, pl.ds(s * RP, RP)])


@functools.partial(
    pl.kernel,
    out_type=jax.ShapeDtypeStruct((NC, NP, D), jnp.float32),
    mesh=_sc_mesh,
    scratch_types=[
        pltpu.VMEM((NCH, C), jnp.int32),
        pltpu.VMEM((NCH, C), jnp.int32),
        pltpu.VMEM((C, D), jnp.float32),
        pltpu.VMEM_SHARED((NP, D), jnp.float32),
        pltpu.SemaphoreType.DMA,
    ],
)
def _spmm_kernel(hs_hbm, src_hbm, dst_hbm, zeros_hbm, out_hbm, srcv, dstv, rows, acc, sem):
    c = lax.axis_index("c")
    s = lax.axis_index("s")
    pltpu.sync_copy(src_hbm.at[c, s], srcv)
    pltpu.sync_copy(dst_hbm.at[c, s], dstv)
    pltpu.sync_copy(zeros_hbm.at[pl.ds(s * RP, RP)], acc.at[pl.ds(s * RP, RP)])
    plsc.subcore_barrier()

    def body(g, carry):
        pltpu.async_copy(hs_hbm.at[srcv.at[g]], rows, sem).wait()
        pltpu.sync_copy(rows, acc.at[dstv.at[g]], add=True)
        return carry

    lax.fori_loop(0, NCH, body, 0)
    plsc.subcore_barrier()
    pltpu.sync_copy(acc.at[pl.ds(s * RP, RP)], out_hbm.at[c, pl.ds(s * RP, RP)])


_BLK = 2000
_GRID = N // _BLK


def _dinv_from(degp_ref):
    deg = 1.0 + degp_ref[0, :, 0] + degp_ref[1, :, 0]
    return lax.rsqrt(deg)


def _stage_a_body(x_ref, wf_ref, bf_ref, degp_ref, h0_ref, h0s_ref):
    h0 = jnp.dot(x_ref[...], wf_ref[...], preferred_element_type=jnp.float32)
    h0 = h0 + bf_ref[...]
    dinv = _dinv_from(degp_ref)
    h0_ref[...] = h0
    h0s_ref[...] = h0 * dinv[:, None]


def _stage_b_body(p_ref, h0_ref, h0s_ref, degp_ref, w_ref, h1_ref, h1s_ref):
    dinv = _dinv_from(degp_ref)
    hi = (p_ref[0] + p_ref[1] + h0s_ref[...]) * dinv[:, None]
    sup = (1.0 - ALPHA) * hi + ALPHA * h0_ref[...]
    h1 = TH1 * jnp.dot(sup, w_ref[...], preferred_element_type=jnp.float32)
    h1 = jnp.maximum(h1 + (1.0 - TH1) * sup, 0.0)
    h1_ref[...] = h1
    h1s_ref[...] = h1 * dinv[:, None]


def _stage_c_body(p_ref, h0_ref, h1s_ref, degp_ref, w_ref, out_ref):
    dinv = _dinv_from(degp_ref)
    hi = (p_ref[0] + p_ref[1] + h1s_ref[...]) * dinv[:, None]
    sup = (1.0 - ALPHA) * hi + ALPHA * h0_ref[...]
    out = TH2 * jnp.dot(sup, w_ref[...], preferred_element_type=jnp.float32)
    out_ref[...] = out + (1.0 - TH2) * sup


_row_spec = pl.BlockSpec((_BLK, D), lambda i: (i, 0))
_p_spec = pl.BlockSpec((NC, _BLK, D), lambda i: (0, i, 0))
_degp_spec = pl.BlockSpec((NC, _BLK, 16), lambda i: (0, i, 0))
_w_spec = pl.BlockSpec((D, D), lambda i: (0, 0))
_b_spec = pl.BlockSpec((1, D), lambda i: (0, 0))
_nd_shape = jax.ShapeDtypeStruct((N, D), jnp.float32)

_stage_a = pl.pallas_call(
    _stage_a_body,
    grid=(_GRID,),
    in_specs=[_row_spec, _w_spec, _b_spec, _degp_spec],
    out_specs=[_row_spec, _row_spec],
    out_shape=[_nd_shape, _nd_shape],
)

_stage_b = pl.pallas_call(
    _stage_b_body,
    grid=(_GRID,),
    in_specs=[_p_spec, _row_spec, _row_spec, _degp_spec, _w_spec],
    out_specs=[_row_spec, _row_spec],
    out_shape=[_nd_shape, _nd_shape],
)

_stage_c = pl.pallas_call(
    _stage_c_body,
    grid=(_GRID,),
    in_specs=[_p_spec, _row_spec, _row_spec, _degp_spec, _w_spec],
    out_specs=_row_spec,
    out_shape=_nd_shape,
)


def kernel(x, edge_index, W_fc, b_fc, W1, W2):
    src = edge_index[0].astype(jnp.int32).reshape(NC, NS, NCH, C)
    dst = edge_index[1].astype(jnp.int32).reshape(NC, NS, NCH, C)
    zeros = jnp.zeros((NP, D), jnp.float32)
    zeros16 = jnp.zeros((NP, 16), jnp.float32)
    ones = jnp.ones((C, 16), jnp.float32)
    b2 = b_fc.reshape(1, D).astype(jnp.float32)

    degp = _deg_kernel(dst, ones, zeros16)
    h0, h0s = _stage_a(x, W_fc, b2, degp)
    p1 = _spmm_kernel(h0s, src, dst, zeros)
    h1, h1s = _stage_b(p1, h0, h0s, degp, W1)
    p2 = _spmm_kernel(h1s, src, dst, zeros)
    return _stage_c(p2, h0, h1s, degp, W2)


# speedup vs baseline: 25.7459x; 1.1815x over previous
import functools
import math

import jax
import jax.numpy as jnp
from jax import lax
from jax.experimental import pallas as pl
from jax.experimental.pallas import tpu as pltpu
from jax.experimental.pallas import tpu_sc as plsc

N = 10000
E = 320000
D = 128
LAMDA = 0.5
ALPHA = 0.1
TH1 = math.log(LAMDA / 1.0 + 1.0)
TH2 = math.log(LAMDA / 2.0 + 1.0)

NC = 2
NS = 16
NCH = 80
C = 125
HB = 40
NP = 10240
RP = NP // NS

_sc_mesh = plsc.VectorSubcoreMesh(
    core_axis_name="c", subcore_axis_name="s", num_cores=NC, num_subcores=NS
)


NCH2, C2 = 125, 80


@functools.partial(
    pl.kernel,
    out_type=jax.ShapeDtypeStruct((NC, NS, RP), jnp.float32),
    mesh=_sc_mesh,
    scratch_types=[
        pltpu.VMEM((NCH2, C2), jnp.int32),
        pltpu.VMEM((C2,), jnp.float32),
        pltpu.VMEM_SHARED((NP,), jnp.float32),
    ],
)
def _deg_kernel(dst_hbm, zerosf_hbm, out_hbm, dstv, onesv, accd):
    c = lax.axis_index("c")
    s = lax.axis_index("s")
    pltpu.sync_copy(dst_hbm.at[c, s], dstv)
    ones16 = jnp.ones((16,), jnp.float32)
    for i in range(C2 // 16):
        onesv[pl.ds(16 * i, 16)] = ones16
    pltpu.sync_copy(zerosf_hbm.at[pl.ds(s * RP, RP)], accd.at[pl.ds(s * RP, RP)])
    plsc.subcore_barrier()

    def body(g, carry):
        pltpu.sync_copy(onesv, accd.at[dstv.at[g]], add=True)
        return carry

    lax.fori_loop(0, NCH2, body, 0)
    plsc.subcore_barrier()
    pltpu.sync_copy(accd.at[pl.ds(s * RP, RP)], out_hbm.at[c, s])


@functools.partial(
    pl.kernel,
    out_type=jax.ShapeDtypeStruct((NC, NP, D), jnp.float32),
    mesh=_sc_mesh,
    scratch_types=[
        pltpu.VMEM((HB, C), jnp.int32),
        pltpu.VMEM((HB, C), jnp.int32),
        pltpu.VMEM((C, D), jnp.float32),
        pltpu.VMEM((C, D), jnp.float32),
        pltpu.VMEM_SHARED((NP, D), jnp.float32),
        pltpu.SemaphoreType.DMA,
        pltpu.SemaphoreType.DMA,
        pltpu.SemaphoreType.DMA,
        pltpu.SemaphoreType.DMA,
    ],
)
def _spmm_kernel(hs_hbm, src_hbm, dst_hbm, zeros_hbm, out_hbm, srcv, dstv,
                 rows0, rows1, acc, semg0, semg1, sems0, sems1):
    c = lax.axis_index("c")
    s = lax.axis_index("s")
    pltpu.sync_copy(zeros_hbm.at[pl.ds(s * RP, RP)], acc.at[pl.ds(s * RP, RP)])
    plsc.subcore_barrier()

    for h in range(NCH // HB):
        pltpu.sync_copy(src_hbm.at[c, s, h], srcv)
        pltpu.sync_copy(dst_hbm.at[c, s, h], dstv)

        def body(t, carry):
            g0 = 2 * t
            d0 = pltpu.async_copy(hs_hbm.at[srcv.at[g0]], rows0, semg0)
            d1 = pltpu.async_copy(hs_hbm.at[srcv.at[g0 + 1]], rows1, semg1)
            d0.wait()
            pltpu.async_copy(rows0, acc.at[dstv.at[g0]], sems0, add=True).wait()
            d1.wait()
            pltpu.async_copy(rows1, acc.at[dstv.at[g0 + 1]], sems1, add=True).wait()
            return carry

        lax.fori_loop(0, HB // 2, body, 0)
    plsc.subcore_barrier()
    pltpu.sync_copy(acc.at[pl.ds(s * RP, RP)], out_hbm.at[c, pl.ds(s * RP, RP)])


_BLK = 1280
_GRID = (N + _BLK - 1) // _BLK


def _dinv_from(degp_ref):
    i = pl.program_id(0)
    deg = 1.0 + degp_ref[0, pl.ds(i * _BLK, _BLK)] + degp_ref[1, pl.ds(i * _BLK, _BLK)]
    return lax.rsqrt(deg)


def _stage_a_body(x_ref, wf_ref, bf_ref, degp_ref, h0_ref, h0s_ref):
    h0 = jnp.dot(x_ref[...], wf_ref[...], preferred_element_type=jnp.float32)
    h0 = h0 + bf_ref[...]
    dinv = _dinv_from(degp_ref)
    h0_ref[...] = h0
    h0s_ref[...] = h0 * dinv[:, None]


def _stage_b_body(p_ref, h0_ref, h0s_ref, degp_ref, w_ref, h1_ref, h1s_ref):
    dinv = _dinv_from(degp_ref)
    hi = (p_ref[0] + p_ref[1] + h0s_ref[...]) * dinv[:, None]
    sup = (1.0 - ALPHA) * hi + ALPHA * h0_ref[...]
    h1 = TH1 * jnp.dot(sup, w_ref[...], preferred_element_type=jnp.float32)
    h1 = jnp.maximum(h1 + (1.0 - TH1) * sup, 0.0)
    h1_ref[...] = h1
    h1s_ref[...] = h1 * dinv[:, None]


def _stage_c_body(p_ref, h0_ref, h1s_ref, degp_ref, w_ref, out_ref):
    dinv = _dinv_from(degp_ref)
    hi = (p_ref[0] + p_ref[1] + h1s_ref[...]) * dinv[:, None]
    sup = (1.0 - ALPHA) * hi + ALPHA * h0_ref[...]
    out = TH2 * jnp.dot(sup, w_ref[...], preferred_element_type=jnp.float32)
    out_ref[...] = out + (1.0 - TH2) * sup


_row_spec = pl.BlockSpec((_BLK, D), lambda i: (i, 0))
_p_spec = pl.BlockSpec((NC, _BLK, D), lambda i: (0, i, 0))
_degp_spec = pl.BlockSpec((NC, NP), lambda i: (0, 0))
_w_spec = pl.BlockSpec((D, D), lambda i: (0, 0))
_b_spec = pl.BlockSpec((1, D), lambda i: (0, 0))
_nd_shape = jax.ShapeDtypeStruct((N, D), jnp.float32)

_stage_a = pl.pallas_call(
    _stage_a_body,
    grid=(_GRID,),
    in_specs=[_row_spec, _w_spec, _b_spec, _degp_spec],
    out_specs=[_row_spec, _row_spec],
    out_shape=[_nd_shape, _nd_shape],
)

_stage_b = pl.pallas_call(
    _stage_b_body,
    grid=(_GRID,),
    in_specs=[_p_spec, _row_spec, _row_spec, _degp_spec, _w_spec],
    out_specs=[_row_spec, _row_spec],
    out_shape=[_nd_shape, _nd_shape],
)

_stage_c = pl.pallas_call(
    _stage_c_body,
    grid=(_GRID,),
    in_specs=[_p_spec, _row_spec, _row_spec, _degp_spec, _w_spec],
    out_specs=_row_spec,
    out_shape=_nd_shape,
)


def kernel(x, edge_index, W_fc, b_fc, W1, W2):
    src = edge_index[0].astype(jnp.int32).reshape(NC, NS, NCH // HB, HB, C)
    dst = edge_index[1].astype(jnp.int32).reshape(NC, NS, NCH // HB, HB, C)
    dst_deg = edge_index[1].astype(jnp.int32).reshape(NC, NS, NCH2, C2)
    zeros = jnp.zeros((NP, D), jnp.float32)
    zerosf = jnp.zeros((NP,), jnp.float32)
    b2 = b_fc.reshape(1, D).astype(jnp.float32)

    degp = _deg_kernel(dst_deg, zerosf).reshape(NC, NP)
    h0, h0s = _stage_a(x, W_fc, b2, degp)
    p1 = _spmm_kernel(h0s, src, dst, zeros)
    h1, h1s = _stage_b(p1, h0, h0s, degp, W1)
    p2 = _spmm_kernel(h1s, src, dst, zeros)
    return _stage_c(p2, h0, h1s, degp, W2)


# speedup vs baseline: 32.7377x; 1.2716x over previous
import functools
import math

import jax
import jax.numpy as jnp
from jax import lax
from jax.experimental import pallas as pl
from jax.experimental.pallas import tpu as pltpu
from jax.experimental.pallas import tpu_sc as plsc

N = 10000
E = 320000
D = 128
LAMDA = 0.5
ALPHA = 0.1
TH1 = math.log(LAMDA / 1.0 + 1.0)
TH2 = math.log(LAMDA / 2.0 + 1.0)

NC = 2
NS = 16
NCH = 80
C = 125
HB = 40
NP = 10240
RP = NP // NS

_sc_mesh = plsc.VectorSubcoreMesh(
    core_axis_name="c", subcore_axis_name="s", num_cores=NC, num_subcores=NS
)


NCH2, C2 = 125, 80


@functools.partial(
    pl.kernel,
    out_type=jax.ShapeDtypeStruct((NC, NS, RP), jnp.float32),
    mesh=_sc_mesh,
    scratch_types=[
        pltpu.VMEM((NCH2, C2), jnp.int32),
        pltpu.VMEM((C2,), jnp.float32),
        pltpu.VMEM_SHARED((NP,), jnp.float32),
    ],
)
def _deg_kernel(dst_hbm, zerosf_hbm, out_hbm, dstv, onesv, accd):
    c = lax.axis_index("c")
    s = lax.axis_index("s")
    pltpu.sync_copy(dst_hbm.at[c, s], dstv)
    ones16 = jnp.ones((16,), jnp.float32)
    for i in range(C2 // 16):
        onesv[pl.ds(16 * i, 16)] = ones16
    pltpu.sync_copy(zerosf_hbm.at[pl.ds(s * RP, RP)], accd.at[pl.ds(s * RP, RP)])
    plsc.subcore_barrier()

    def body(g, carry):
        pltpu.sync_copy(onesv, accd.at[dstv.at[g]], add=True)
        return carry

    lax.fori_loop(0, NCH2, body, 0)
    plsc.subcore_barrier()
    pltpu.sync_copy(accd.at[pl.ds(s * RP, RP)], out_hbm.at[c, s])


@functools.partial(
    pl.kernel,
    out_type=jax.ShapeDtypeStruct((NC, NP, D), jnp.float32),
    mesh=_sc_mesh,
    scratch_types=[
        pltpu.VMEM((HB, C), jnp.int32),
        pltpu.VMEM((HB, C), jnp.int32),
        pltpu.VMEM((C, D), jnp.float32),
        pltpu.VMEM((C, D), jnp.float32),
        pltpu.VMEM_SHARED((NP, D), jnp.float32),
        pltpu.SemaphoreType.DMA,
        pltpu.SemaphoreType.DMA,
        pltpu.SemaphoreType.DMA,
        pltpu.SemaphoreType.DMA,
    ],
)
def _spmm_kernel(hs_hbm, src_hbm, dst_hbm, zeros_hbm, out_hbm, srcv, dstv,
                 rows0, rows1, acc, semg0, semg1, sems0, sems1):
    c = lax.axis_index("c")
    s = lax.axis_index("s")
    pltpu.sync_copy(zeros_hbm.at[pl.ds(s * RP, RP)], acc.at[pl.ds(s * RP, RP)])
    plsc.subcore_barrier()

    for h in range(NCH // HB):
        pltpu.sync_copy(src_hbm.at[c, s, h], srcv)
        pltpu.sync_copy(dst_hbm.at[c, s, h], dstv)

        pltpu.async_copy(hs_hbm.at[srcv.at[0]], rows0, semg0)

        def body(t, carry):
            g0 = 2 * t
            pltpu.async_copy(hs_hbm.at[srcv.at[g0 + 1]], rows1, semg1)
            pltpu.make_async_copy(hs_hbm.at[srcv.at[g0]], rows0, semg0).wait()
            pltpu.async_copy(rows0, acc.at[dstv.at[g0]], sems0, add=True).wait()

            @pl.when(g0 + 2 < HB)
            def _():
                pltpu.async_copy(hs_hbm.at[srcv.at[g0 + 2]], rows0, semg0)

            pltpu.make_async_copy(hs_hbm.at[srcv.at[g0 + 1]], rows1, semg1).wait()
            pltpu.async_copy(rows1, acc.at[dstv.at[g0 + 1]], sems1, add=True).wait()
            return carry

        lax.fori_loop(0, HB // 2, body, 0)
    plsc.subcore_barrier()
    pltpu.sync_copy(acc.at[pl.ds(s * RP, RP)], out_hbm.at[c, pl.ds(s * RP, RP)])


_BLK = 1280
_GRID = (N + _BLK - 1) // _BLK


def _dinv_from(degp_ref):
    i = pl.program_id(0)
    deg = 1.0 + degp_ref[0, pl.ds(i * _BLK, _BLK)] + degp_ref[1, pl.ds(i * _BLK, _BLK)]
    return lax.rsqrt(deg)


def _stage_a_body(x_ref, wf_ref, bf_ref, degp_ref, h0_ref, h0s_ref):
    h0 = jnp.dot(x_ref[...], wf_ref[...], preferred_element_type=jnp.float32)
    h0 = h0 + bf_ref[...]
    dinv = _dinv_from(degp_ref)
    h0_ref[...] = h0
    h0s_ref[...] = h0 * dinv[:, None]


def _stage_b_body(p_ref, h0_ref, h0s_ref, degp_ref, w_ref, h1_ref, h1s_ref):
    dinv = _dinv_from(degp_ref)
    hi = (p_ref[0] + p_ref[1] + h0s_ref[...]) * dinv[:, None]
    sup = (1.0 - ALPHA) * hi + ALPHA * h0_ref[...]
    h1 = TH1 * jnp.dot(sup, w_ref[...], preferred_element_type=jnp.float32)
    h1 = jnp.maximum(h1 + (1.0 - TH1) * sup, 0.0)
    h1_ref[...] = h1
    h1s_ref[...] = h1 * dinv[:, None]


def _stage_c_body(p_ref, h0_ref, h1s_ref, degp_ref, w_ref, out_ref):
    dinv = _dinv_from(degp_ref)
    hi = (p_ref[0] + p_ref[1] + h1s_ref[...]) * dinv[:, None]
    sup = (1.0 - ALPHA) * hi + ALPHA * h0_ref[...]
    out = TH2 * jnp.dot(sup, w_ref[...], preferred_element_type=jnp.float32)
    out_ref[...] = out + (1.0 - TH2) * sup


_row_spec = pl.BlockSpec((_BLK, D), lambda i: (i, 0))
_p_spec = pl.BlockSpec((NC, _BLK, D), lambda i: (0, i, 0))
_degp_spec = pl.BlockSpec((NC, NP), lambda i: (0, 0))
_w_spec = pl.BlockSpec((D, D), lambda i: (0, 0))
_b_spec = pl.BlockSpec((1, D), lambda i: (0, 0))
_nd_shape = jax.ShapeDtypeStruct((N, D), jnp.float32)

_stage_a = pl.pallas_call(
    _stage_a_body,
    grid=(_GRID,),
    in_specs=[_row_spec, _w_spec, _b_spec, _degp_spec],
    out_specs=[_row_spec, _row_spec],
    out_shape=[_nd_shape, _nd_shape],
)

_stage_b = pl.pallas_call(
    _stage_b_body,
    grid=(_GRID,),
    in_specs=[_p_spec, _row_spec, _row_spec, _degp_spec, _w_spec],
    out_specs=[_row_spec, _row_spec],
    out_shape=[_nd_shape, _nd_shape],
)

_stage_c = pl.pallas_call(
    _stage_c_body,
    grid=(_GRID,),
    in_specs=[_p_spec, _row_spec, _row_spec, _degp_spec, _w_spec],
    out_specs=_row_spec,
    out_shape=_nd_shape,
)


def kernel(x, edge_index, W_fc, b_fc, W1, W2):
    src = edge_index[0].astype(jnp.int32).reshape(NC, NS, NCH // HB, HB, C)
    dst = edge_index[1].astype(jnp.int32).reshape(NC, NS, NCH // HB, HB, C)
    dst_deg = edge_index[1].astype(jnp.int32).reshape(NC, NS, NCH2, C2)
    zeros = jnp.zeros((NP, D), jnp.float32)
    zerosf = jnp.zeros((NP,), jnp.float32)
    b2 = b_fc.reshape(1, D).astype(jnp.float32)

    degp = _deg_kernel(dst_deg, zerosf).reshape(NC, NP)
    h0, h0s = _stage_a(x, W_fc, b2, degp)
    p1 = _spmm_kernel(h0s, src, dst, zeros)
    h1, h1s = _stage_b(p1, h0, h0s, degp, W1)
    p2 = _spmm_kernel(h1s, src, dst, zeros)
    return _stage_c(p2, h0, h1s, degp, W2)


# speedup vs baseline: 34.3227x; 1.0484x over previous
import functools
import math

import jax
import jax.numpy as jnp
from jax import lax
from jax.experimental import pallas as pl
from jax.experimental.pallas import tpu as pltpu
from jax.experimental.pallas import tpu_sc as plsc

N = 10000
E = 320000
D = 128
LAMDA = 0.5
ALPHA = 0.1
TH1 = math.log(LAMDA / 1.0 + 1.0)
TH2 = math.log(LAMDA / 2.0 + 1.0)

NC = 2
NS = 16
NCH = 80
C = 125
HB = 40
NP = 10240
RP = NP // NS

_sc_mesh = plsc.VectorSubcoreMesh(
    core_axis_name="c", subcore_axis_name="s", num_cores=NC, num_subcores=NS
)


NCH2, C2 = 125, 80


@functools.partial(
    pl.kernel,
    out_type=jax.ShapeDtypeStruct((NC, NS, RP), jnp.float32),
    mesh=_sc_mesh,
    scratch_types=[
        pltpu.VMEM((NCH2, C2), jnp.int32),
        pltpu.VMEM((C2,), jnp.float32),
        pltpu.VMEM_SHARED((NP,), jnp.float32),
        pltpu.SemaphoreType.DMA,
    ],
)
def _deg_kernel(dst_hbm, zerosf_hbm, out_hbm, dstv, onesv, accd, sem):
    c = lax.axis_index("c")
    s = lax.axis_index("s")
    pltpu.sync_copy(dst_hbm.at[c, s], dstv)
    ones16 = jnp.ones((16,), jnp.float32)
    for i in range(C2 // 16):
        onesv[pl.ds(16 * i, 16)] = ones16
    pltpu.sync_copy(zerosf_hbm.at[pl.ds(s * RP, RP)], accd.at[pl.ds(s * RP, RP)])
    plsc.subcore_barrier()

    def body(g, carry):
        pltpu.async_copy(onesv, accd.at[dstv.at[g]], sem, add=True)
        return carry

    lax.fori_loop(0, NCH2, body, 0)

    def drain(g, carry):
        pltpu.make_async_copy(onesv, accd.at[dstv.at[0]], sem).wait()
        return carry

    lax.fori_loop(0, NCH2, drain, 0)
    plsc.subcore_barrier()
    pltpu.sync_copy(accd.at[pl.ds(s * RP, RP)], out_hbm.at[c, s])


@functools.partial(
    pl.kernel,
    out_type=jax.ShapeDtypeStruct((NC, NP, D), jnp.float32),
    mesh=_sc_mesh,
    scratch_types=[
        pltpu.VMEM((HB, C), jnp.int32),
        pltpu.VMEM((HB, C), jnp.int32),
        pltpu.VMEM((C, D), jnp.float32),
        pltpu.VMEM((C, D), jnp.float32),
        pltpu.VMEM_SHARED((NP, D), jnp.float32),
        pltpu.SemaphoreType.DMA,
        pltpu.SemaphoreType.DMA,
        pltpu.SemaphoreType.DMA,
        pltpu.SemaphoreType.DMA,
    ],
)
def _spmm_kernel(hs_hbm, src_hbm, dst_hbm, zeros_hbm, out_hbm, srcv, dstv,
                 rows0, rows1, acc, semg0, semg1, sems0, sems1):
    c = lax.axis_index("c")
    s = lax.axis_index("s")
    dz = pltpu.async_copy(zeros_hbm.at[pl.ds(s * RP, RP)], acc.at[pl.ds(s * RP, RP)], sems0)

    for h in range(NCH // HB):
        pltpu.sync_copy(src_hbm.at[c, s, h], srcv)
        pltpu.sync_copy(dst_hbm.at[c, s, h], dstv)

        pltpu.async_copy(hs_hbm.at[srcv.at[0]], rows0, semg0)
        if h == 0:
            dz.wait()
            plsc.subcore_barrier()

        def body(t, carry):
            g0 = 2 * t
            pltpu.async_copy(hs_hbm.at[srcv.at[g0 + 1]], rows1, semg1)
            pltpu.make_async_copy(hs_hbm.at[srcv.at[g0]], rows0, semg0).wait()
            pltpu.async_copy(rows0, acc.at[dstv.at[g0]], sems0, add=True).wait()

            @pl.when(g0 + 2 < HB)
            def _():
                pltpu.async_copy(hs_hbm.at[srcv.at[g0 + 2]], rows0, semg0)

            pltpu.make_async_copy(hs_hbm.at[srcv.at[g0 + 1]], rows1, semg1).wait()
            pltpu.async_copy(rows1, acc.at[dstv.at[g0 + 1]], sems1, add=True).wait()
            return carry

        lax.fori_loop(0, HB // 2, body, 0)
    plsc.subcore_barrier()
    pltpu.sync_copy(acc.at[pl.ds(s * RP, RP)], out_hbm.at[c, pl.ds(s * RP, RP)])


_BLK = 1280
_GRID = (N + _BLK - 1) // _BLK


def _dinv_from(degp_ref):
    i = pl.program_id(0)
    deg = 1.0 + degp_ref[0, pl.ds(i * _BLK, _BLK)] + degp_ref[1, pl.ds(i * _BLK, _BLK)]
    return lax.rsqrt(deg)


def _stage_a_body(x_ref, wf_ref, bf_ref, degp_ref, h0_ref, h0s_ref):
    h0 = jnp.dot(x_ref[...], wf_ref[...], preferred_element_type=jnp.float32)
    h0 = h0 + bf_ref[...]
    dinv = _dinv_from(degp_ref)
    h0_ref[...] = h0
    h0s_ref[...] = h0 * dinv[:, None]


def _stage_b_body(p_ref, h0_ref, h0s_ref, degp_ref, w_ref, h1_ref, h1s_ref):
    dinv = _dinv_from(degp_ref)
    hi = (p_ref[0] + p_ref[1] + h0s_ref[...]) * dinv[:, None]
    sup = (1.0 - ALPHA) * hi + ALPHA * h0_ref[...]
    h1 = TH1 * jnp.dot(sup, w_ref[...], preferred_element_type=jnp.float32)
    h1 = jnp.maximum(h1 + (1.0 - TH1) * sup, 0.0)
    h1_ref[...] = h1
    h1s_ref[...] = h1 * dinv[:, None]


def _stage_c_body(p_ref, h0_ref, h1s_ref, degp_ref, w_ref, out_ref):
    dinv = _dinv_from(degp_ref)
    hi = (p_ref[0] + p_ref[1] + h1s_ref[...]) * dinv[:, None]
    sup = (1.0 - ALPHA) * hi + ALPHA * h0_ref[...]
    out = TH2 * jnp.dot(sup, w_ref[...], preferred_element_type=jnp.float32)
    out_ref[...] = out + (1.0 - TH2) * sup


_row_spec = pl.BlockSpec((_BLK, D), lambda i: (i, 0))
_p_spec = pl.BlockSpec((NC, _BLK, D), lambda i: (0, i, 0))
_degp_spec = pl.BlockSpec((NC, NP), lambda i: (0, 0))
_w_spec = pl.BlockSpec((D, D), lambda i: (0, 0))
_b_spec = pl.BlockSpec((1, D), lambda i: (0, 0))
_nd_shape = jax.ShapeDtypeStruct((N, D), jnp.float32)

_stage_a = pl.pallas_call(
    _stage_a_body,
    grid=(_GRID,),
    in_specs=[_row_spec, _w_spec, _b_spec, _degp_spec],
    out_specs=[_row_spec, _row_spec],
    out_shape=[_nd_shape, _nd_shape],
)

_stage_b = pl.pallas_call(
    _stage_b_body,
    grid=(_GRID,),
    in_specs=[_p_spec, _row_spec, _row_spec, _degp_spec, _w_spec],
    out_specs=[_row_spec, _row_spec],
    out_shape=[_nd_shape, _nd_shape],
)

_stage_c = pl.pallas_call(
    _stage_c_body,
    grid=(_GRID,),
    in_specs=[_p_spec, _row_spec, _row_spec, _degp_spec, _w_spec],
    out_specs=_row_spec,
    out_shape=_nd_shape,
)


def kernel(x, edge_index, W_fc, b_fc, W1, W2):
    src = edge_index[0].astype(jnp.int32).reshape(NC, NS, NCH // HB, HB, C)
    dst = edge_index[1].astype(jnp.int32).reshape(NC, NS, NCH // HB, HB, C)
    dst_deg = edge_index[1].astype(jnp.int32).reshape(NC, NS, NCH2, C2)
    zeros = jnp.zeros((NP, D), jnp.float32)
    zerosf = jnp.zeros((NP,), jnp.float32)
    b2 = b_fc.reshape(1, D).astype(jnp.float32)

    degp = _deg_kernel(dst_deg, zerosf).reshape(NC, NP)
    h0, h0s = _stage_a(x, W_fc, b2, degp)
    p1 = _spmm_kernel(h0s, src, dst, zeros)
    h1, h1s = _stage_b(p1, h0, h0s, degp, W1)
    p2 = _spmm_kernel(h1s, src, dst, zeros)
    return _stage_c(p2, h0, h1s, degp, W2)
